# baseline (speedup 1.0000x reference)
import functools

import jax
import jax.numpy as jnp
from jax import lax
from jax.experimental import pallas as pl
from jax.experimental.pallas import tpu as pltpu
from jax.experimental.pallas import tpu_sc as plsc

EMBED = 64
NW = 32
IDX_ROW = 128
GATHERS_PER_GROUP = 4
ROWS_PER_GROUP = IDX_ROW * GATHERS_PER_GROUP
NBUF = 2


@jax.jit
def _sc_embedding_gather(table, idx_flat):
    B = idx_flat.shape[0]
    b_per_w = B // NW
    n_idx_rows = b_per_w // IDX_ROW
    n_groups = b_per_w // ROWS_PER_GROUP
    assert n_groups % NBUF == 0
    idx3 = idx_flat.reshape(NW, n_idx_rows, IDX_ROW)

    mesh = plsc.VectorSubcoreMesh(core_axis_name="c", subcore_axis_name="s")

    @functools.partial(
        pl.kernel,
        mesh=mesh,
        compiler_params=pltpu.CompilerParams(use_tc_tiling_on_sc=False),
        out_type=jax.ShapeDtypeStruct((B, EMBED), jnp.float32),
        scratch_types=[
            pltpu.VMEM((n_idx_rows, IDX_ROW), jnp.int32),
            pltpu.VMEM((NBUF, ROWS_PER_GROUP, EMBED), jnp.float32),
            pltpu.SemaphoreType.DMA,
            pltpu.SemaphoreType.DMA,
        ],
    )
    def k(table_hbm, idx_hbm, out_hbm, idx_v, rows_v, gsem, osem):
        cid = lax.axis_index("c")
        sid = lax.axis_index("s")
        wid = sid * 2 + cid
        base = wid * b_per_w
        pltpu.sync_copy(idx_hbm.at[wid], idx_v)

        def run_group(g, buf):
            copies = [
                pltpu.async_copy(
                    table_hbm.at[idx_v.at[g * GATHERS_PER_GROUP + j]],
                    rows_v.at[buf].at[pl.ds(j * IDX_ROW, IDX_ROW)],
                    gsem,
                )
                for j in range(GATHERS_PER_GROUP)
            ]
            for c in copies:
                c.wait()
            pltpu.async_copy(
                rows_v.at[buf],
                out_hbm.at[pl.ds(base + g * ROWS_PER_GROUP, ROWS_PER_GROUP)],
                osem,
            )

        def wait_one_out():
            pltpu.make_async_copy(
                rows_v.at[0],
                out_hbm.at[pl.ds(base, ROWS_PER_GROUP)],
                osem,
            ).wait()

        for buf in range(NBUF):
            run_group(buf, buf)

        @pl.loop(NBUF, n_groups, step=NBUF)
        def _(g0):
            for buf in range(NBUF):
                wait_one_out()
                run_group(g0 + buf, buf)

        for _ in range(NBUF):
            wait_one_out()

    return k(table, idx3)


def kernel(x, table):
    b, f = x.shape
    idx_flat = x.reshape(b * f)
    out = _sc_embedding_gather(table, idx_flat)
    return out.reshape(b, f, EMBED)
